# Initial kernel scaffold; baseline (speedup 1.0000x reference)
import jax
import jax.numpy as jnp
from jax.experimental import pallas as pl


def kernel(x, res2, out, W, b):
    raise NotImplementedError("write your pallas kernel here")


# speedup vs baseline: 1.5223x; 1.5223x over previous
import functools

import jax
import jax.numpy as jnp
from jax.experimental import pallas as pl
from jax.experimental.pallas import tpu as pltpu
from jax.experimental.pallas import tpu_sc as plsc


def _coords(c, n):
    g = 2.0 * c - 1.0
    v = ((g + 1.0) * n - 1.0) / 2.0
    v0 = jnp.floor(v)
    v1 = v0 + 1.0
    w1 = v - v0
    w0 = 1.0 - w1
    return v0, v1, w0, w1


def _b1_body(outf_ref, og_ref, cov_ref, pts_ref, idx_ref):
    outf = outf_ref[0]
    og = og_ref[0]
    cov = cov_ref[0]

    m0 = jnp.maximum(outf[0:1], outf[1:2])
    m1 = jnp.minimum(outf[0:1], outf[1:2])

    x = og[:, 0:1]
    y = og[:, 1:2]
    x0, x1, wx0, wx1 = _coords(x, 32)
    y0, y1, wy0, wy1 = _coords(y, 32)
    iota = jax.lax.broadcasted_iota(jnp.int32, (96, 1024), 1)

    def gat(ix, iy, m):
        valid = (ix >= 0) & (ix < 32) & (iy >= 0) & (iy < 32)
        ixc = jnp.clip(ix, 0, 31).astype(jnp.int32)
        iyc = jnp.clip(iy, 0, 31).astype(jnp.int32)
        oh = ((iota == iyc * 32 + ixc) & valid).astype(jnp.float32)
        return jnp.sum(oh * m, axis=1, keepdims=True)

    def samp(m):
        return (gat(x0, y0, m) * (wx0 * wy0)
                + gat(x1, y0, m) * (wx1 * wy0)
                + gat(x0, y1, m) * (wx0 * wy1)
                + gat(x1, y1, m) * (wx1 * wy1))

    u_col = -1.0 * (samp(m0) - samp(m1))

    ii = jax.lax.broadcasted_iota(jnp.int32, (96, 96), 0)
    jj = jax.lax.broadcasted_iota(jnp.int32, (96, 96), 1)
    eye = ii == jj
    u_row = jnp.sum(jnp.where(eye, u_col, 0.0), axis=0, keepdims=True)
    beats = (u_row > u_col) | ((u_row == u_col) & (jj < ii))
    rank_col = jnp.sum(beats.astype(jnp.int32), axis=1, keepdims=True)
    rank_row = jnp.sum(jnp.where(eye, rank_col, 0), axis=0, keepdims=True)

    sel = (jax.lax.broadcasted_iota(jnp.int32, (24, 96), 0) == rank_row)
    selq = sel.astype(jnp.float32)
    x_row = jnp.sum(jnp.where(eye, x, 0.0), axis=0, keepdims=True)
    y_row = jnp.sum(jnp.where(eye, y, 0.0), axis=0, keepdims=True)
    imp_x = jnp.sum(selq * x_row, axis=1, keepdims=True)
    imp_y = jnp.sum(selq * y_row, axis=1, keepdims=True)
    imp = jnp.concatenate([imp_x, imp_y], axis=1)
    pts = jnp.concatenate([imp, cov], axis=0)
    pts_ref[0] = pts

    fx0, fx1, _, _ = _coords(pts[:, 0:1], 128)
    fy0, fy1, _, _ = _coords(pts[:, 1:2], 128)
    x0i = jnp.clip(fx0, 0, 127).astype(jnp.int32)
    x1i = jnp.clip(fx1, 0, 127).astype(jnp.int32)
    y0i = jnp.clip(fy0, 0, 127).astype(jnp.int32)
    y1i = jnp.clip(fy1, 0, 127).astype(jnp.int32)

    r = jax.lax.broadcasted_iota(jnp.int32, (1, 128), 1)
    p_of_r = r >> 2
    n_of_r = r & 3
    p32 = jax.lax.broadcasted_iota(jnp.int32, (32, 128), 0) == p_of_r

    def row128(col):
        return jnp.sum(jnp.where(p32, col, 0), axis=0, keepdims=True)

    ysel = jnp.where(n_of_r < 2, row128(y0i), row128(y1i))
    xsel = jnp.where((n_of_r & 1) == 0, row128(x0i), row128(x1i))
    rowval = ysel * 8 + (xsel >> 4)
    b = pl.program_id(0)
    cc = jax.lax.broadcasted_iota(jnp.int32, (512, 128), 0)
    idx_ref[0] = (cc + b * 512) * 1024 + rowval


def _make_sc_gather(rows):
    nw = 32
    per_w = rows // nw
    ch = per_w // 2
    mesh = plsc.VectorSubcoreMesh(core_axis_name="c", subcore_axis_name="s")

    @functools.partial(
        pl.kernel, mesh=mesh,
        compiler_params=pltpu.CompilerParams(use_tc_tiling_on_sc=False),
        out_type=jax.ShapeDtypeStruct((rows, 16), jnp.float32),
        scratch_types=[
            pltpu.VMEM((ch,), jnp.int32),
            pltpu.VMEM((ch, 16), jnp.float32),
            pltpu.SemaphoreType.DMA,
        ],
    )
    def sc_gather(table_hbm, idx_hbm, out_hbm, idx_v, rows_v, sem):
        wid = jax.lax.axis_index("s") * 2 + jax.lax.axis_index("c")
        for rr in range(2):
            base = wid * per_w + rr * ch
            pltpu.sync_copy(idx_hbm.at[pl.ds(base, ch)], idx_v)
            pltpu.async_copy(table_hbm.at[idx_v], rows_v, sem).wait()
            pltpu.sync_copy(rows_v, out_hbm.at[pl.ds(base, ch)])

    return sc_gather


def _b2_body(g_ref, pts_ref, outf_ref, wc_ref, wf_ref, bias_ref, rend_ref):
    G = g_ref[0]
    pts = pts_ref[0]
    outf = outf_ref[0]
    wc = wc_ref[...]
    wf = wf_ref[...]

    x = pts[:, 0:1]
    y = pts[:, 1:2]
    fx0, fx1, wx0, wx1 = _coords(x, 128)
    fy0, fy1, wy0, wy1 = _coords(y, 128)
    vx0 = ((fx0 >= 0) & (fx0 < 128)).astype(jnp.float32)
    vx1 = ((fx1 >= 0) & (fx1 < 128)).astype(jnp.float32)
    vy0 = ((fy0 >= 0) & (fy0 < 128)).astype(jnp.float32)
    vy1 = ((fy1 >= 0) & (fy1 < 128)).astype(jnp.float32)
    offx0 = jnp.clip(fx0, 0, 127).astype(jnp.int32) & 15
    offx1 = jnp.clip(fx1, 0, 127).astype(jnp.int32) & 15
    w00 = wx0 * wy0 * (vx0 * vy0)
    w10 = wx1 * wy0 * (vx1 * vy0)
    w01 = wx0 * wy1 * (vx0 * vy1)
    w11 = wx1 * wy1 * (vx1 * vy1)

    q = jax.lax.broadcasted_iota(jnp.int32, (1, 2048), 1)
    off_q = q & 15
    n_q = (q >> 4) & 3
    p_q = q >> 6
    p32 = jax.lax.broadcasted_iota(jnp.int32, (32, 2048), 0) == p_q

    def rowq_f(col):
        return jnp.sum(jnp.where(p32, col, 0.0), axis=0, keepdims=True)

    def rowq_i(col):
        return jnp.sum(jnp.where(p32, col, 0), axis=0, keepdims=True)

    offsel = jnp.where((n_q & 1) == 0, rowq_i(offx0), rowq_i(offx1))
    wsel = jnp.where(n_q == 0, rowq_f(w00),
                     jnp.where(n_q == 1, rowq_f(w10),
                               jnp.where(n_q == 2, rowq_f(w01), rowq_f(w11))))
    msk = (off_q == offsel).astype(jnp.float32) * wsel

    contrib = G * msk
    A = jnp.dot(wf, contrib, preferred_element_type=jnp.float32)
    grp = ((jax.lax.broadcasted_iota(jnp.int32, (2048, 32), 0) >> 6)
           == jax.lax.broadcasted_iota(jnp.int32, (2048, 32), 1))
    rend_fine = jnp.dot(A, grp.astype(jnp.float32),
                        preferred_element_type=jnp.float32)

    projout = wc[:, 0:1] * outf[0:1] + wc[:, 1:2] * outf[1:2]
    eye32 = (jax.lax.broadcasted_iota(jnp.int32, (32, 32), 0)
             == jax.lax.broadcasted_iota(jnp.int32, (32, 32), 1))
    x_row = jnp.sum(jnp.where(eye32, x, 0.0), axis=0, keepdims=True)
    y_row = jnp.sum(jnp.where(eye32, y, 0.0), axis=0, keepdims=True)
    cx0, cx1, cwx0, cwx1 = _coords(x_row, 32)
    cy0, cy1, cwy0, cwy1 = _coords(y_row, 32)
    iota = jax.lax.broadcasted_iota(jnp.int32, (1024, 32), 0)

    def gatc(ix, iy):
        valid = (ix >= 0) & (ix < 32) & (iy >= 0) & (iy < 32)
        ixc = jnp.clip(ix, 0, 31).astype(jnp.int32)
        iyc = jnp.clip(iy, 0, 31).astype(jnp.int32)
        oh = ((iota == iyc * 32 + ixc) & valid).astype(jnp.float32)
        return jnp.dot(projout, oh, preferred_element_type=jnp.float32)

    coarse = (gatc(cx0, cy0) * (cwx0 * cwy0)
              + gatc(cx1, cy0) * (cwx1 * cwy0)
              + gatc(cx0, cy1) * (cwx0 * cwy1)
              + gatc(cx1, cy1) * (cwx1 * cwy1))

    rend_ref[0] = coarse + rend_fine + bias_ref[...]


def kernel(x, res2, out, W, b):
    B, Cr, Hr, Wr = res2.shape
    N = x.shape[-1] // 16
    kN = 3 * N
    bN = int(0.75 * N)

    key = jax.random.key(42)
    k1, k2 = jax.random.split(key)
    over_gen = jax.random.uniform(k1, (B, kN, 2), dtype=out.dtype)
    coverage = jax.random.uniform(k2, (B, N - bN, 2), dtype=out.dtype)

    outf = out.reshape(B, 2, 32 * 32)
    wf = W[:, 2:]
    wc = W[:, :2]
    bias = b.reshape(2, 1)

    pts, idx = pl.pallas_call(
        _b1_body,
        grid=(B,),
        in_specs=[
            pl.BlockSpec((1, 2, 1024), lambda bb: (bb, 0, 0)),
            pl.BlockSpec((1, kN, 2), lambda bb: (bb, 0, 0)),
            pl.BlockSpec((1, N - bN, 2), lambda bb: (bb, 0, 0)),
        ],
        out_specs=[
            pl.BlockSpec((1, N, 2), lambda bb: (bb, 0, 0)),
            pl.BlockSpec((1, Cr, 4 * N), lambda bb: (bb, 0, 0)),
        ],
        out_shape=[
            jax.ShapeDtypeStruct((B, N, 2), jnp.float32),
            jax.ShapeDtypeStruct((B, Cr, 4 * N), jnp.int32),
        ],
    )(outf, over_gen, coverage)

    rows = B * Cr * 4 * N
    G = _make_sc_gather(rows)(res2.reshape(B * Cr * Hr * Wr // 16, 16),
                              idx.reshape(rows))

    rend = pl.pallas_call(
        _b2_body,
        grid=(B,),
        in_specs=[
            pl.BlockSpec((1, Cr, 4 * N * 16), lambda bb: (bb, 0, 0)),
            pl.BlockSpec((1, N, 2), lambda bb: (bb, 0, 0)),
            pl.BlockSpec((1, 2, 1024), lambda bb: (bb, 0, 0)),
            pl.BlockSpec((2, 2), lambda bb: (0, 0)),
            pl.BlockSpec((2, Cr), lambda bb: (0, 0)),
            pl.BlockSpec((2, 1), lambda bb: (0, 0)),
        ],
        out_specs=pl.BlockSpec((1, 2, N), lambda bb: (bb, 0, 0)),
        out_shape=jax.ShapeDtypeStruct((B, 2, N), jnp.float32),
    )(G.reshape(B, Cr, 4 * N * 16), pts, outf, wc, wf, bias)

    return (rend, pts)
